# baseline (speedup 1.0000x reference)
import functools
import math

import jax
import jax.numpy as jnp
import numpy as np
from jax import lax
from jax.experimental import pallas as pl


def _pick_split(L: int):
    n2 = min(128, L // 2)
    return L // n2, n2


def _tables(L: int):
    N1, N2 = _pick_split(L)
    a1 = np.arange(N1, dtype=np.int64)
    a2 = np.arange(N2, dtype=np.int64)
    ang1 = (np.outer(a1, a1) % N1).astype(np.float64) * (2.0 * np.pi / N1)
    ang2 = (np.outer(a2, a2) % N2).astype(np.float64) * (2.0 * np.pi / N2)
    w1c, w1s = np.cos(ang1), np.sin(ang1)
    w2c, w2s = np.cos(ang2), np.sin(ang2)
    angt = (np.outer(a2, a1) % L).astype(np.float64) * (2.0 * np.pi / L)
    twc, tws = np.cos(angt), np.sin(angt)

    f32 = lambda x: x.astype(np.float32)
    mats = dict(
        w1f=f32(np.concatenate([w1c, -w1s], axis=1)),
        w2f=f32(np.block([[w2c, -w2s], [w2s, w2c]])),
        w2i=f32(np.block([[w2c, w2s], [-w2s, w2c]])),
        w1i=f32(np.concatenate([w1c, -w1s], axis=0) / L),
    )
    tabs = {}
    for name, m in mats.items():
        hi = m.astype(jnp.bfloat16)
        lo = (m - np.asarray(hi, np.float32)).astype(jnp.bfloat16)
        tabs[name + "_hi"] = jnp.asarray(hi)
        tabs[name + "_3"] = jnp.asarray(np.concatenate([hi, lo, hi], axis=0))
    tabs["twct"] = jnp.asarray(f32(twc.T[None]))
    tabs["twst"] = jnp.asarray(f32(tws.T[None]))
    return N1, N2, tabs


_BF = jnp.bfloat16


def _bdot(a, b):
    return jnp.dot(a, b, preferred_element_type=jnp.float32)


def _hi_lo(x):
    hi = x.astype(_BF)
    lo = (x - hi.astype(jnp.float32)).astype(_BF)
    return hi, lo


def _fft_fwd(x, t, precise):
    M, N1, N2 = x.shape
    if precise:
        x_hi, x_lo = _hi_lo(x)
        xht = jnp.swapaxes(x_hi, 1, 2)
        xlt = jnp.swapaxes(x_lo, 1, 2)
        lhs = jnp.concatenate([xht, xht, xlt], axis=2).reshape(M * N2, 3 * N1)
        a = _bdot(lhs, t["w1f_3"])
    else:
        xt = jnp.swapaxes(x.astype(_BF), 1, 2).reshape(M * N2, N1)
        a = _bdot(xt, t["w1f_hi"])
    at = jnp.swapaxes(a.reshape(M, N2, 2 * N1), 1, 2)
    are, aim = at[:, :N1, :], at[:, N1:, :]
    bre = are * t["twct"] + aim * t["twst"]
    bim = aim * t["twct"] - are * t["twst"]
    if precise:
        bre_hi, bre_lo = _hi_lo(bre)
        bim_hi, bim_lo = _hi_lo(bim)
        lhs2 = jnp.concatenate(
            [bre_hi, bim_hi, bre_hi, bim_hi, bre_lo, bim_lo],
            axis=2).reshape(M * N1, 6 * N2)
        c = _bdot(lhs2, t["w2f_3"])
    else:
        lhs2 = jnp.concatenate([bre.astype(_BF), bim.astype(_BF)],
                               axis=2).reshape(M * N1, 2 * N2)
        c = _bdot(lhs2, t["w2f_hi"])
    c = c.reshape(M, N1, 2 * N2)
    return c[:, :, :N2], c[:, :, N2:]


def _fft_inv_real(zre, zim, t, precise):
    M, N1, N2 = zre.shape
    if precise:
        zre_hi, zre_lo = _hi_lo(zre)
        zim_hi, zim_lo = _hi_lo(zim)
        lhs = jnp.concatenate(
            [zre_hi, zim_hi, zre_hi, zim_hi, zre_lo, zim_lo],
            axis=2).reshape(M * N1, 6 * N2)
        g = _bdot(lhs, t["w2i_3"])
    else:
        lhs = jnp.concatenate([zre.astype(_BF), zim.astype(_BF)],
                              axis=2).reshape(M * N1, 2 * N2)
        g = _bdot(lhs, t["w2i_hi"])
    g = g.reshape(M, N1, 2 * N2)
    gre, gim = g[:, :, :N2], g[:, :, N2:]
    hre = gre * t["twct"] - gim * t["twst"]
    him = gre * t["twst"] + gim * t["twct"]
    if precise:
        hre_hi, hre_lo = _hi_lo(hre)
        him_hi, him_lo = _hi_lo(him)
        h_hi = jnp.swapaxes(jnp.concatenate([hre_hi, him_hi], axis=1), 1, 2)
        h_lo = jnp.swapaxes(jnp.concatenate([hre_lo, him_lo], axis=1), 1, 2)
        lhs2 = jnp.concatenate([h_hi, h_hi, h_lo],
                               axis=2).reshape(M * N2, 6 * N1)
        o = _bdot(lhs2, t["w1i_3"])
    else:
        hcat = jnp.concatenate([hre.astype(_BF), him.astype(_BF)], axis=1)
        lhs2 = jnp.swapaxes(hcat, 1, 2).reshape(M * N2, 2 * N1)
        o = _bdot(lhs2, t["w1i_hi"])
    return jnp.swapaxes(o.reshape(M, N2, N1), 1, 2)


def _autocorr_body(k_top, C, L, N1, N2, tab_names, q_ref, k_ref, v_ref,
                   *tab_refs_and_out):
    tab_refs = tab_refs_and_out[:-1]
    o_ref = tab_refs_and_out[-1]
    t = {name: ref[...] for name, ref in zip(tab_names, tab_refs)}
    f32 = jnp.float32

    q_re, q_im = _fft_fwd(q_ref[...], t, precise=True)
    k_re, k_im = _fft_fwd(k_ref[...], t, precise=True)
    z_re = q_re * k_re + q_im * k_im
    z_im = q_im * k_re - q_re * k_im
    corr = _fft_inv_real(z_re, z_im, t, precise=True)

    tau = (N2 * lax.broadcasted_iota(jnp.int32, (C, N1, N2), 1)
           + lax.broadcasted_iota(jnp.int32, (C, N1, N2), 2))
    vals, taos = [], []
    c = corr
    for _ in range(k_top):
        m_i = jnp.max(jnp.max(c, axis=1, keepdims=True), axis=2,
                      keepdims=True)
        hit = jnp.where(c == m_i, tau, L)
        first = jnp.min(jnp.min(hit, axis=1, keepdims=True), axis=2,
                        keepdims=True)
        vals.append(m_i)
        taos.append(first)
        c = jnp.where(tau == first, -jnp.inf, c)

    exps = [jnp.exp(x_ - vals[0]) for x_ in vals]
    denom = exps[0]
    for e in exps[1:]:
        denom = denom + e

    s = jnp.zeros((C, N1, N2), f32)
    for i in range(k_top):
        s = s + jnp.where(tau == taos[i], exps[i] / denom, f32(0.0))

    s_re, s_im = _fft_fwd(s, t, precise=False)
    v_re, v_im = _fft_fwd(v_ref[...], t, precise=False)
    o_re = v_re * s_re + v_im * s_im
    o_im = v_im * s_re - v_re * s_im
    o_ref[...] = _fft_inv_real(o_re, o_im, t, precise=False)


def kernel(queries, keys, values, attn_mask):
    B, L, H, E = queries.shape
    NCH = B * H * E
    k_top = int(math.log(L))

    N1, N2, tabs = _tables(L)
    tab_names = list(tabs.keys())
    tab_vals = [tabs[n] for n in tab_names]

    def pre(x):
        return x.transpose(0, 2, 3, 1).reshape(NCH, N1, N2)

    q2, k2, v2 = pre(queries), pre(keys), pre(values)

    C = 256
    while NCH % C:
        C //= 2
    grid = NCH // C

    body = functools.partial(_autocorr_body, k_top, C, L, N1, N2, tab_names)

    row_spec = pl.BlockSpec((C, N1, N2), lambda i: (i, 0, 0))
    tab_specs = [
        pl.BlockSpec(tv.shape, lambda i, n=tv.ndim: (0,) * n)
        for tv in tab_vals
    ]
    out2 = pl.pallas_call(
        body,
        grid=(grid,),
        in_specs=[row_spec, row_spec, row_spec] + tab_specs,
        out_specs=row_spec,
        out_shape=jax.ShapeDtypeStruct((NCH, N1, N2), jnp.float32),
    )(q2, k2, v2, *tab_vals)

    return out2.reshape(B, H, E, L).transpose(0, 3, 1, 2)


# speedup vs baseline: 1.1354x; 1.1354x over previous
import functools
import math

import jax
import jax.numpy as jnp
import numpy as np
from jax import lax
from jax.experimental import pallas as pl


def _pick_split(L: int):
    n2 = min(128, L // 2)
    return L // n2, n2


def _tables(L: int):
    N1, N2 = _pick_split(L)
    a1 = np.arange(N1, dtype=np.int64)
    a2 = np.arange(N2, dtype=np.int64)
    ang1 = (np.outer(a1, a1) % N1).astype(np.float64) * (2.0 * np.pi / N1)
    ang2 = (np.outer(a2, a2) % N2).astype(np.float64) * (2.0 * np.pi / N2)
    w1c, w1s = np.cos(ang1), np.sin(ang1)
    w2c, w2s = np.cos(ang2), np.sin(ang2)
    angt = (np.outer(a2, a1) % L).astype(np.float64) * (2.0 * np.pi / L)
    twc, tws = np.cos(angt), np.sin(angt)

    f32 = lambda x: x.astype(np.float32)
    mats = dict(
        w1f=f32(np.concatenate([w1c, -w1s], axis=1)),
        w2f=f32(np.block([[w2c, -w2s], [w2s, w2c]])),
        w2i=f32(np.block([[w2c, w2s], [-w2s, w2c]])),
        w1i=f32(np.concatenate([w1c, -w1s], axis=0) / L),
    )
    tabs = {}
    for name, m in mats.items():
        hi = m.astype(jnp.bfloat16)
        lo = (m - np.asarray(hi, np.float32)).astype(jnp.bfloat16)
        tabs[name + "_hi"] = jnp.asarray(hi)
        tabs[name + "_3"] = jnp.asarray(np.concatenate([hi, lo, hi], axis=0))
    tabs["twct"] = jnp.asarray(f32(twc.T[None]))
    tabs["twst"] = jnp.asarray(f32(tws.T[None]))
    return N1, N2, tabs


_BF = jnp.bfloat16


def _bdot(a, b):
    return jnp.dot(a, b, preferred_element_type=jnp.float32)


def _hi_lo(x):
    hi = x.astype(_BF)
    lo = (x - hi.astype(jnp.float32)).astype(_BF)
    return hi, lo


def _fft_fwd(x, t, precise):
    M, N1, N2 = x.shape
    if precise:
        x_hi, x_lo = _hi_lo(x)
        xht = jnp.swapaxes(x_hi, 1, 2)
        xlt = jnp.swapaxes(x_lo, 1, 2)
        lhs = jnp.concatenate([xht, xht, xlt], axis=2).reshape(M * N2, 3 * N1)
        a = _bdot(lhs, t["w1f_3"])
    else:
        xt = jnp.swapaxes(x.astype(_BF), 1, 2).reshape(M * N2, N1)
        a = _bdot(xt, t["w1f_hi"])
    at = jnp.swapaxes(a.reshape(M, N2, 2 * N1), 1, 2)
    are, aim = at[:, :N1, :], at[:, N1:, :]
    bre = are * t["twct"] + aim * t["twst"]
    bim = aim * t["twct"] - are * t["twst"]
    if precise:
        bre_hi, bre_lo = _hi_lo(bre)
        bim_hi, bim_lo = _hi_lo(bim)
        lhs2 = jnp.concatenate(
            [bre_hi, bim_hi, bre_hi, bim_hi, bre_lo, bim_lo],
            axis=2).reshape(M * N1, 6 * N2)
        c = _bdot(lhs2, t["w2f_3"])
    else:
        lhs2 = jnp.concatenate([bre.astype(_BF), bim.astype(_BF)],
                               axis=2).reshape(M * N1, 2 * N2)
        c = _bdot(lhs2, t["w2f_hi"])
    c = c.reshape(M, N1, 2 * N2)
    return c[:, :, :N2], c[:, :, N2:]


def _fft_inv_real(zre, zim, t, precise):
    M, N1, N2 = zre.shape
    if precise:
        zre_hi, zre_lo = _hi_lo(zre)
        zim_hi, zim_lo = _hi_lo(zim)
        lhs = jnp.concatenate(
            [zre_hi, zim_hi, zre_hi, zim_hi, zre_lo, zim_lo],
            axis=2).reshape(M * N1, 6 * N2)
        g = _bdot(lhs, t["w2i_3"])
    else:
        lhs = jnp.concatenate([zre.astype(_BF), zim.astype(_BF)],
                              axis=2).reshape(M * N1, 2 * N2)
        g = _bdot(lhs, t["w2i_hi"])
    g = g.reshape(M, N1, 2 * N2)
    gre, gim = g[:, :, :N2], g[:, :, N2:]
    hre = gre * t["twct"] - gim * t["twst"]
    him = gre * t["twst"] + gim * t["twct"]
    if precise:
        hre_hi, hre_lo = _hi_lo(hre)
        him_hi, him_lo = _hi_lo(him)
        h_hi = jnp.swapaxes(jnp.concatenate([hre_hi, him_hi], axis=1), 1, 2)
        h_lo = jnp.swapaxes(jnp.concatenate([hre_lo, him_lo], axis=1), 1, 2)
        lhs2 = jnp.concatenate([h_hi, h_hi, h_lo],
                               axis=2).reshape(M * N2, 6 * N1)
        o = _bdot(lhs2, t["w1i_3"])
    else:
        hcat = jnp.concatenate([hre.astype(_BF), him.astype(_BF)], axis=1)
        lhs2 = jnp.swapaxes(hcat, 1, 2).reshape(M * N2, 2 * N1)
        o = _bdot(lhs2, t["w1i_hi"])
    return jnp.swapaxes(o.reshape(M, N2, N1), 1, 2)


def _autocorr_body(k_top, C, L, N1, N2, tab_names, q_ref, k_ref, v_ref,
                   *tab_refs_and_out):
    tab_refs = tab_refs_and_out[:-1]
    o_ref = tab_refs_and_out[-1]
    t = {name: ref[...] for name, ref in zip(tab_names, tab_refs)}
    f32 = jnp.float32

    q_re, q_im = _fft_fwd(q_ref[...], t, precise=True)
    k_re, k_im = _fft_fwd(k_ref[...], t, precise=True)
    z_re = q_re * k_re + q_im * k_im
    z_im = q_im * k_re - q_re * k_im
    corr = _fft_inv_real(z_re, z_im, t, precise=True)

    tau = (N2 * lax.broadcasted_iota(jnp.int32, (C, N1, N2), 1)
           + lax.broadcasted_iota(jnp.int32, (C, N1, N2), 2))
    vals, taos = [], []
    c = corr
    for _ in range(k_top):
        m_i = jnp.max(jnp.max(c, axis=1, keepdims=True), axis=2,
                      keepdims=True)
        hit = jnp.where(c == m_i, tau, L)
        first = jnp.min(jnp.min(hit, axis=1, keepdims=True), axis=2,
                        keepdims=True)
        vals.append(m_i)
        taos.append(first)
        c = jnp.where(tau == first, -jnp.inf, c)

    exps = [jnp.exp(x_ - vals[0]) for x_ in vals]
    denom = exps[0]
    for e in exps[1:]:
        denom = denom + e

    s = jnp.zeros((C, N1, N2), f32)
    for i in range(k_top):
        s = s + jnp.where(tau == taos[i], exps[i] / denom, f32(0.0))

    s_re, s_im = _fft_fwd(s, t, precise=False)
    v_re, v_im = _fft_fwd(v_ref[...], t, precise=False)
    o_re = v_re * s_re + v_im * s_im
    o_im = v_im * s_re - v_re * s_im
    o_ref[...] = _fft_inv_real(o_re, o_im, t, precise=False)


def kernel(queries, keys, values, attn_mask):
    B, L, H, E = queries.shape
    NCH = B * H * E
    k_top = int(math.log(L))

    N1, N2, tabs = _tables(L)
    tab_names = list(tabs.keys())
    tab_vals = [tabs[n] for n in tab_names]

    def pre(x):
        return x.transpose(0, 2, 3, 1).reshape(NCH, N1, N2)

    q2, k2, v2 = pre(queries), pre(keys), pre(values)

    C = 128
    while NCH % C:
        C //= 2
    grid = NCH // C

    body = functools.partial(_autocorr_body, k_top, C, L, N1, N2, tab_names)

    row_spec = pl.BlockSpec((C, N1, N2), lambda i: (i, 0, 0))
    tab_specs = [
        pl.BlockSpec(tv.shape, lambda i, n=tv.ndim: (0,) * n)
        for tv in tab_vals
    ]
    out2 = pl.pallas_call(
        body,
        grid=(grid,),
        in_specs=[row_spec, row_spec, row_spec] + tab_specs,
        out_specs=row_spec,
        out_shape=jax.ShapeDtypeStruct((NCH, N1, N2), jnp.float32),
    )(q2, k2, v2, *tab_vals)

    return out2.reshape(B, H, E, L).transpose(0, 3, 1, 2)
